# baseline (speedup 1.0000x reference)
import functools

import jax
import jax.numpy as jnp
from jax import lax
from jax.experimental import pallas as pl
from jax.experimental.pallas import tpu as pltpu
from jax.experimental.pallas import tpu_sc as plsc

_H = 14
_W = 14
_HW = _H * _W


def _sc_gather(idx, table):
    B = idx.shape[0]
    D = table.shape[1]
    info = plsc.get_sparse_core_info()
    nw = info.num_cores * info.num_subcores
    b_per_w = B // nw
    mesh = plsc.VectorSubcoreMesh(core_axis_name="c", subcore_axis_name="s")

    @functools.partial(
        pl.kernel,
        mesh=mesh,
        out_type=jax.ShapeDtypeStruct((B, D), jnp.float32),
        scratch_types=[
            pltpu.VMEM((b_per_w,), jnp.int32),
            pltpu.VMEM((b_per_w, D), jnp.float32),
            pltpu.SemaphoreType.DMA,
        ],
        compiler_params=pltpu.CompilerParams(use_tc_tiling_on_sc=False),
    )
    def k(idx_hbm, table_hbm, out_hbm, idx_v, rows_v, sem):
        wid = lax.axis_index("s") * info.num_cores + lax.axis_index("c")
        base = wid * b_per_w
        pltpu.sync_copy(idx_hbm.at[pl.ds(base, b_per_w)], idx_v)
        pltpu.async_copy(table_hbm.at[idx_v], rows_v, sem).wait()
        pltpu.sync_copy(rows_v, out_hbm.at[pl.ds(base, b_per_w)])

    return k(idx, table)


def _tc_transpose(emb):
    B, D = emb.shape

    def body(e_ref, o_ref):
        o_ref[...] = jnp.transpose(e_ref[...])

    return pl.pallas_call(
        body,
        out_shape=jax.ShapeDtypeStruct((D, B), jnp.float32),
    )(emb)


def _tc_broadcast(emb_t):
    D, B = emb_t.shape
    nsem = 8

    def body(e_ref, o_ref, sems):
        handles = []
        for hw in range(_HW):
            if hw >= nsem:
                handles[hw - nsem].wait()
            handles.append(
                pltpu.async_copy(e_ref, o_ref.at[hw], sems.at[hw % nsem]))
        for hw in range(_HW - nsem, _HW):
            handles[hw].wait()

    return pl.pallas_call(
        body,
        in_specs=[pl.BlockSpec((D, B), lambda: (0, 0))],
        out_specs=pl.BlockSpec(memory_space=pl.ANY),
        out_shape=jax.ShapeDtypeStruct((_HW, D, B), jnp.float32),
        scratch_shapes=[pltpu.SemaphoreType.DMA((nsem,))],
    )(emb_t)


def kernel(labels, table):
    B = labels.shape[0]
    D = table.shape[1]
    labels = labels.astype(jnp.int32)
    emb = _sc_gather(labels, table)
    emb_t = _tc_transpose(emb)
    out = _tc_broadcast(emb_t)
    return out.reshape(_H, _W, D, B).transpose(3, 2, 0, 1)


# speedup vs baseline: 1.0666x; 1.0666x over previous
import functools

import jax
import jax.numpy as jnp
from jax import lax
from jax.experimental import pallas as pl
from jax.experimental.pallas import tpu as pltpu
from jax.experimental.pallas import tpu_sc as plsc

_H = 14
_W = 14
_HW = _H * _W


def _sc_gather(idx, table128):
    B = idx.shape[0]
    D2 = table128.shape[1]
    info = plsc.get_sparse_core_info()
    nw = info.num_cores * info.num_subcores
    b_per_w = B // nw
    mesh = plsc.VectorSubcoreMesh(core_axis_name="c", subcore_axis_name="s")

    @functools.partial(
        pl.kernel,
        mesh=mesh,
        out_type=jax.ShapeDtypeStruct((B, D2), jnp.float32),
        scratch_types=[
            pltpu.VMEM((b_per_w,), jnp.int32),
            pltpu.VMEM((b_per_w, D2), jnp.float32),
            pltpu.SemaphoreType.DMA,
        ],
    )
    def k(idx_hbm, table_hbm, out_hbm, idx_v, rows_v, sem):
        wid = lax.axis_index("s") * info.num_cores + lax.axis_index("c")
        base = wid * b_per_w
        pltpu.sync_copy(idx_hbm.at[pl.ds(base, b_per_w)], idx_v)
        pltpu.async_copy(table_hbm.at[idx_v], rows_v, sem).wait()
        pltpu.sync_copy(rows_v, out_hbm.at[pl.ds(base, b_per_w)])

    return k(idx, table128)


def _tc_transpose(wide, D):
    B = wide.shape[0]

    def body(w_ref, o_ref):
        o_ref[...] = jnp.transpose(w_ref[...])[:D]

    return pl.pallas_call(
        body,
        out_shape=jax.ShapeDtypeStruct((D, B), jnp.float32),
    )(wide)


def _tc_broadcast(emb_t):
    D, B = emb_t.shape
    nsem = 8

    def body(e_ref, o_ref, sems):
        handles = []
        for hw in range(_HW):
            if hw >= nsem:
                handles[hw - nsem].wait()
            handles.append(
                pltpu.async_copy(e_ref, o_ref.at[hw], sems.at[hw % nsem]))
        for hw in range(_HW - nsem, _HW):
            handles[hw].wait()

    return pl.pallas_call(
        body,
        in_specs=[pl.BlockSpec((D, B), lambda: (0, 0))],
        out_specs=pl.BlockSpec(memory_space=pl.ANY),
        out_shape=jax.ShapeDtypeStruct((_HW, D, B), jnp.float32),
        scratch_shapes=[pltpu.SemaphoreType.DMA((nsem,))],
    )(emb_t)


def kernel(labels, table):
    B = labels.shape[0]
    V, D = table.shape
    labels = labels.astype(jnp.int32)
    table128 = jnp.pad(table, ((0, 0), (0, 128 - D)))
    wide = _sc_gather(labels, table128)
    emb_t = _tc_transpose(wide, D)
    out = _tc_broadcast(emb_t)
    return out.reshape(_H, _W, D, B).transpose(3, 2, 0, 1)


# speedup vs baseline: 1.3309x; 1.2478x over previous
import functools

import jax
import jax.numpy as jnp
from jax import lax
from jax.experimental import pallas as pl
from jax.experimental.pallas import tpu as pltpu
from jax.experimental.pallas import tpu_sc as plsc

_H = 14
_W = 14
_HW = _H * _W


def _sc_gather(idx, table128):
    B = idx.shape[0]
    D2 = table128.shape[1]
    info = plsc.get_sparse_core_info()
    nw = info.num_cores * info.num_subcores
    b_per_w = B // nw
    mesh = plsc.VectorSubcoreMesh(core_axis_name="c", subcore_axis_name="s")

    @functools.partial(
        pl.kernel,
        mesh=mesh,
        out_type=jax.ShapeDtypeStruct((B, D2), jnp.float32),
        scratch_types=[
            pltpu.VMEM((b_per_w,), jnp.int32),
            pltpu.VMEM((b_per_w, D2), jnp.float32),
            pltpu.SemaphoreType.DMA,
        ],
    )
    def k(idx_hbm, table_hbm, out_hbm, idx_v, rows_v, sem):
        wid = lax.axis_index("s") * info.num_cores + lax.axis_index("c")
        base = wid * b_per_w
        pltpu.sync_copy(idx_hbm.at[pl.ds(base, b_per_w)], idx_v)
        pltpu.async_copy(table_hbm.at[idx_v], rows_v, sem).wait()
        pltpu.sync_copy(rows_v, out_hbm.at[pl.ds(base, b_per_w)])

    return k(idx, table128)


def _tc_convert(table_t):
    D, V = table_t.shape
    lb = 12800

    def body(t_ref, o_ref):
        wt = jnp.transpose(t_ref[...])
        o_ref[...] = jnp.concatenate(
            [wt, jnp.zeros((lb, D), jnp.float32)], axis=1)

    return pl.pallas_call(
        body,
        grid=(pl.cdiv(V, lb),),
        in_specs=[pl.BlockSpec((D, lb), lambda i: (0, i))],
        out_specs=pl.BlockSpec((lb, 2 * D), lambda i: (i, 0)),
        out_shape=jax.ShapeDtypeStruct((V, 2 * D), jnp.float32),
    )(table_t)


def _tc_transpose(wide, D):
    B = wide.shape[0]

    def body(w_ref, o_ref):
        o_ref[...] = jnp.transpose(w_ref[...])[:D]

    return pl.pallas_call(
        body,
        out_shape=jax.ShapeDtypeStruct((D, B), jnp.float32),
    )(wide)


def _tc_broadcast(emb_t):
    D, B = emb_t.shape
    nsem = 8

    def body(e_ref, o_ref, sems):
        handles = []
        for hw in range(_HW):
            if hw >= nsem:
                handles[hw - nsem].wait()
            handles.append(
                pltpu.async_copy(e_ref, o_ref.at[hw], sems.at[hw % nsem]))
        for hw in range(_HW - nsem, _HW):
            handles[hw].wait()

    return pl.pallas_call(
        body,
        in_specs=[pl.BlockSpec((D, B), lambda: (0, 0))],
        out_specs=pl.BlockSpec(memory_space=pl.ANY),
        out_shape=jax.ShapeDtypeStruct((_HW, D, B), jnp.float32),
        scratch_shapes=[pltpu.SemaphoreType.DMA((nsem,))],
    )(emb_t)


def kernel(labels, table):
    B = labels.shape[0]
    V, D = table.shape
    labels = labels.astype(jnp.int32)
    table128 = _tc_convert(jnp.transpose(table))
    wide = _sc_gather(labels, table128)
    emb_t = _tc_transpose(wide, D)
    out = _tc_broadcast(emb_t)
    return out.reshape(_H, _W, D, B).transpose(3, 2, 0, 1)


# speedup vs baseline: 1.3592x; 1.0213x over previous
import functools

import jax
import jax.numpy as jnp
from jax import lax
from jax.experimental import pallas as pl
from jax.experimental.pallas import tpu as pltpu
from jax.experimental.pallas import tpu_sc as plsc

_H = 14
_W = 14
_HW = _H * _W


_LB = 12800
_HB = _LB // 2


def _tc_convert(table_t):
    D, V = table_t.shape
    nblk = pl.cdiv(V, _LB)

    def body(t_ref, o_ref):
        wt = jnp.transpose(t_ref[...])
        o_ref[...] = jnp.concatenate([wt[:_HB], wt[_HB:]], axis=1)

    return pl.pallas_call(
        body,
        grid=(nblk,),
        in_specs=[pl.BlockSpec((D, _LB), lambda i: (0, i))],
        out_specs=pl.BlockSpec((_HB, 2 * D), lambda i: (i, 0)),
        out_shape=jax.ShapeDtypeStruct((nblk * _HB, 2 * D), jnp.float32),
    )(table_t)


def _sc_gather(idx, packed):
    B = idx.shape[0]
    D2 = packed.shape[1]
    info = plsc.get_sparse_core_info()
    nw = info.num_cores * info.num_subcores
    b_per_w = B // nw
    mesh = plsc.VectorSubcoreMesh(core_axis_name="c", subcore_axis_name="s")

    @functools.partial(
        pl.kernel,
        mesh=mesh,
        out_type=jax.ShapeDtypeStruct((B, D2), jnp.float32),
        scratch_types=[
            pltpu.VMEM((b_per_w,), jnp.int32),
            pltpu.VMEM((b_per_w, D2), jnp.float32),
            pltpu.SemaphoreType.DMA,
        ],
    )
    def k(idx_hbm, table_hbm, out_hbm, idx_v, rows_v, sem):
        wid = lax.axis_index("s") * info.num_cores + lax.axis_index("c")
        base = wid * b_per_w
        pltpu.sync_copy(idx_hbm.at[pl.ds(base, b_per_w)], idx_v)
        pltpu.async_copy(table_hbm.at[idx_v], rows_v, sem).wait()
        pltpu.sync_copy(rows_v, out_hbm.at[pl.ds(base, b_per_w)])

    return k(idx, packed)


def _tc_select_broadcast(wide, parity, D):
    B = wide.shape[0]
    nsem = 8

    def body(w_ref, p_ref, o_ref, e_ref, sems):
        wt = jnp.transpose(w_ref[...])
        par = p_ref[...] > 0
        e_ref[...] = jnp.where(par, wt[D:], wt[:D])
        handles = []
        for hw in range(_HW):
            if hw >= nsem:
                handles[hw - nsem].wait()
            handles.append(
                pltpu.async_copy(e_ref, o_ref.at[hw], sems.at[hw % nsem]))
        for hw in range(_HW - nsem, _HW):
            handles[hw].wait()

    return pl.pallas_call(
        body,
        in_specs=[
            pl.BlockSpec((B, 2 * D), lambda: (0, 0)),
            pl.BlockSpec((1, B), lambda: (0, 0)),
        ],
        out_specs=pl.BlockSpec(memory_space=pl.ANY),
        out_shape=jax.ShapeDtypeStruct((_HW, D, B), jnp.float32),
        scratch_shapes=[
            pltpu.VMEM((D, B), jnp.float32),
            pltpu.SemaphoreType.DMA((nsem,)),
        ],
    )(wide, parity)


def kernel(labels, table):
    B = labels.shape[0]
    D = table.shape[1]
    labels = labels.astype(jnp.int32)
    packed = _tc_convert(jnp.transpose(table))
    j = labels % _LB
    idx = (labels // _LB) * _HB + (j % _HB)
    half = j // _HB
    wide = _sc_gather(idx, packed)
    parity = half.reshape(1, B)
    out = _tc_select_broadcast(wide, parity, D)
    return out.reshape(_H, _W, D, B).transpose(3, 2, 0, 1)
